# Initial kernel scaffold; baseline (speedup 1.0000x reference)
import jax
import jax.numpy as jnp
from jax.experimental import pallas as pl


def kernel(x, edge_index, edge_attr, target_dist, Wl1, bl1, Wr1, Wl2, bl2, Wr2, Wl3, bl3, Wr3, We1, be1, We2, be2, We3, be3):
    raise NotImplementedError("write your pallas kernel here")


# speedup vs baseline: 3.5218x; 3.5218x over previous
import functools

import jax
import jax.numpy as jnp
from jax import lax
from jax.experimental import pallas as pl
from jax.experimental.pallas import tpu as pltpu
from jax.experimental.pallas import tpu_sc as plsc

N = 10000
NP = 10240
E = 160000
NC = 2
NS = 16
RPT = NP // NS

_MESH = dict(core_axis_name="c", subcore_axis_name="s", num_cores=NC,
             num_subcores=NS)


def _sc_mesh():
    return plsc.VectorSubcoreMesh(**_MESH)


K1 = 200
EPT1 = E // (NC * NS)


def _sc_agg16(xp, src, dst, z16):
    @functools.partial(
        pl.kernel,
        out_type=jax.ShapeDtypeStruct((NC, NP, 128), jnp.float32),
        mesh=_sc_mesh(),
        scratch_types=[
            pltpu.VMEM((K1,), jnp.int32),
            pltpu.VMEM((K1,), jnp.int32),
            pltpu.VMEM((K1, 128), jnp.float32),
            pltpu.VMEM_SHARED((NP, 128), jnp.float32),
            pltpu.SemaphoreType.DMA,
        ],
    )
    def k(xp_hbm, src_hbm, dst_hbm, z_hbm, out_hbm, idx_s, idx_d, rows, acc,
          sem):
        c = lax.axis_index("c")
        s = lax.axis_index("s")
        pltpu.sync_copy(z_hbm.at[pl.ds(s * RPT, RPT)],
                        acc.at[pl.ds(s * RPT, RPT)])
        plsc.subcore_barrier()
        base = (c * NS + s) * EPT1

        def chunk(j, carry):
            off = base + j * K1
            pltpu.sync_copy(src_hbm.at[pl.ds(off, K1)], idx_s)
            pltpu.sync_copy(dst_hbm.at[pl.ds(off, K1)], idx_d)
            pltpu.async_copy(xp_hbm.at[idx_s], rows, sem).wait()
            pltpu.sync_copy(rows, acc.at[idx_d], add=True)
            return carry

        lax.fori_loop(0, EPT1 // K1, chunk, 0)
        plsc.subcore_barrier()
        pltpu.sync_copy(acc.at[pl.ds(s * RPT, RPT)],
                        out_hbm.at[c, pl.ds(s * RPT, RPT)])

    return k(xp, src, dst, z16)


K2 = 200
EPT2 = E // NS


def _sc_agg128(h_split, src, dst, z128):
    @functools.partial(
        pl.kernel,
        out_type=jax.ShapeDtypeStruct((NC, NP, 128), jnp.float32),
        mesh=_sc_mesh(),
        scratch_types=[
            pltpu.VMEM((K2,), jnp.int32),
            pltpu.VMEM((K2,), jnp.int32),
            pltpu.VMEM((K2, 128), jnp.float32),
            pltpu.VMEM_SHARED((NP, 128), jnp.float32),
            pltpu.SemaphoreType.DMA,
        ],
    )
    def k(h_hbm, src_hbm, dst_hbm, z_hbm, out_hbm, idx_s, idx_d, rows, acc,
          sem):
        c = lax.axis_index("c")
        s = lax.axis_index("s")
        pltpu.sync_copy(z_hbm.at[pl.ds(s * RPT, RPT)],
                        acc.at[pl.ds(s * RPT, RPT)])
        plsc.subcore_barrier()
        base = s * EPT2

        def chunk(j, carry):
            off = base + j * K2
            pltpu.sync_copy(src_hbm.at[pl.ds(off, K2)], idx_s)
            pltpu.sync_copy(dst_hbm.at[pl.ds(off, K2)], idx_d)
            pltpu.async_copy(h_hbm.at[c].at[idx_s], rows, sem).wait()
            pltpu.sync_copy(rows, acc.at[idx_d], add=True)
            return carry

        lax.fori_loop(0, EPT2 // K2, chunk, 0)
        plsc.subcore_barrier()
        pltpu.sync_copy(acc.at[pl.ds(s * RPT, RPT)],
                        out_hbm.at[c, pl.ds(s * RPT, RPT)])

    return k(h_split, src, dst, z128)


K3 = 400


def _sc_edge_gather(a_split, b_split, row, col):
    out_t = jax.ShapeDtypeStruct((NC, E, 128), jnp.float32)

    @functools.partial(
        pl.kernel,
        out_type=(out_t, out_t),
        mesh=_sc_mesh(),
        scratch_types=[
            pltpu.VMEM((K3,), jnp.int32),
            pltpu.VMEM((K3, 128), jnp.float32),
            pltpu.SemaphoreType.DMA,
        ],
    )
    def k(a_hbm, b_hbm, row_hbm, col_hbm, ga_hbm, gb_hbm, idx, rows, sem):
        c = lax.axis_index("c")
        s = lax.axis_index("s")
        base = s * EPT2

        def chunk(j, carry):
            off = base + j * K3
            pltpu.sync_copy(row_hbm.at[pl.ds(off, K3)], idx)
            pltpu.async_copy(a_hbm.at[c].at[idx], rows, sem).wait()
            pltpu.sync_copy(rows, ga_hbm.at[c, pl.ds(off, K3)])
            pltpu.sync_copy(col_hbm.at[pl.ds(off, K3)], idx)
            pltpu.async_copy(b_hbm.at[c].at[idx], rows, sem).wait()
            pltpu.sync_copy(rows, gb_hbm.at[c, pl.ds(off, K3)])
            return carry

        lax.fori_loop(0, EPT2 // K3, chunk, 0)

    return k(a_split, b_split, row, col)


BM = 1024


def _full(shape):
    return pl.BlockSpec(shape, lambda i: tuple(0 for _ in shape))


def _tc_layer1(part, xp, wl_t, bl, wr_t):
    def body(p_ref, xp_ref, wl_ref, bl_ref, wr_ref, out_ref):
        agg = p_ref[0] + p_ref[1]
        cnt = jnp.maximum(agg[:, 3:4], 1.0)
        mean = agg / cnt
        h = jnp.dot(mean, wl_ref[...], preferred_element_type=jnp.float32)
        h += jnp.dot(xp_ref[...], wr_ref[...],
                     preferred_element_type=jnp.float32)
        h = jnp.maximum(h + bl_ref[...], 0.0)
        out_ref[0] = h[:, :128]
        out_ref[1] = h[:, 128:]

    return pl.pallas_call(
        body,
        grid=(NP // BM,),
        in_specs=[
            pl.BlockSpec((NC, BM, 128), lambda i: (0, i, 0)),
            pl.BlockSpec((BM, 128), lambda i: (i, 0)),
            _full((128, 256)),
            _full((1, 256)),
            _full((128, 256)),
        ],
        out_specs=pl.BlockSpec((NC, BM, 128), lambda i: (0, i, 0)),
        out_shape=jax.ShapeDtypeStruct((NC, NP, 128), jnp.float32),
    )(part, xp, wl_t, bl, wr_t)


def _tc_layer(agg_split, part, h_prev, wl_t, bl, wr_t, ab=None):
    def body(a_ref, p_ref, h_ref, wl_ref, bl_ref, wr_ref, *rest):
        cnt = jnp.maximum(p_ref[0][:, 3:4] + p_ref[1][:, 3:4], 1.0)
        mean = jnp.concatenate([a_ref[0], a_ref[1]], axis=1) / cnt
        hp = jnp.concatenate([h_ref[0], h_ref[1]], axis=1)
        h = jnp.dot(mean, wl_ref[...], preferred_element_type=jnp.float32)
        h += jnp.dot(hp, wr_ref[...], preferred_element_type=jnp.float32)
        h = jnp.maximum(h + bl_ref[...], 0.0)
        if ab is None:
            out_ref = rest[0]
            out_ref[0] = h[:, :128]
            out_ref[1] = h[:, 128:]
        else:
            wa_ref, wb_ref, oa_ref, ob_ref = rest
            a = jnp.dot(h, wa_ref[...], preferred_element_type=jnp.float32)
            b = jnp.dot(h, wb_ref[...], preferred_element_type=jnp.float32)
            oa_ref[0] = a[:, :128]
            oa_ref[1] = a[:, 128:]
            ob_ref[0] = b[:, :128]
            ob_ref[1] = b[:, 128:]

    split_spec = pl.BlockSpec((NC, BM, 128), lambda i: (0, i, 0))
    in_specs = [
        split_spec,
        pl.BlockSpec((NC, BM, 128), lambda i: (0, i, 0)),
        split_spec,
        _full((256, 256)),
        _full((1, 256)),
        _full((256, 256)),
    ]
    split_shape = jax.ShapeDtypeStruct((NC, NP, 128), jnp.float32)
    args = [agg_split, part, h_prev, wl_t, bl, wr_t]
    if ab is None:
        out_specs, out_shape = split_spec, split_shape
    else:
        in_specs += [_full((256, 256)), _full((256, 256))]
        args += [ab[0], ab[1]]
        out_specs = (split_spec, split_spec)
        out_shape = (split_shape, split_shape)
    return pl.pallas_call(
        body, grid=(NP // BM,), in_specs=in_specs, out_specs=out_specs,
        out_shape=out_shape,
    )(*args)


BE = 1280


def _tc_edge_mlp(ga, gb, ea, td, wattr, wtd, be1, w2_t, be2, w3, be3):
    def body(ga_ref, gb_ref, ea_ref, td_ref, wattr_ref, wtd_ref, be1_ref,
             w2_ref, be2_ref, w3_ref, be3_ref, out_ref):
        g = jnp.concatenate(
            [ga_ref[0] + gb_ref[0], ga_ref[1] + gb_ref[1]], axis=1)
        c0 = be1_ref[...] + td_ref[0, 0] * wtd_ref[...]
        z1 = jnp.maximum(g + ea_ref[...] * wattr_ref[...] + c0, 0.0)
        z2 = jnp.dot(z1, w2_ref[...], preferred_element_type=jnp.float32)
        z2 = jnp.maximum(z2 + be2_ref[...], 0.0)
        out_ref[...] = (jnp.sum(z2 * w3_ref[...], axis=1, keepdims=True)
                        + be3_ref[0, 0])

    return pl.pallas_call(
        body,
        grid=(E // BE,),
        in_specs=[
            pl.BlockSpec((NC, BE, 128), lambda i: (0, i, 0)),
            pl.BlockSpec((NC, BE, 128), lambda i: (0, i, 0)),
            pl.BlockSpec((BE, 1), lambda i: (i, 0)),
            _full((1, 1)),
            _full((1, 256)),
            _full((1, 256)),
            _full((1, 256)),
            _full((256, 128)),
            _full((1, 128)),
            _full((1, 128)),
            _full((1, 1)),
        ],
        out_specs=pl.BlockSpec((BE, 1), lambda i: (i, 0)),
        out_shape=jax.ShapeDtypeStruct((E, 1), jnp.float32),
    )(ga, gb, ea, td, wattr, wtd, be1, w2_t, be2, w3, be3)


def kernel(x, edge_index, edge_attr, target_dist,
           Wl1, bl1, Wr1, Wl2, bl2, Wr2, Wl3, bl3, Wr3,
           We1, be1, We2, be2, We3, be3):
    f32 = jnp.float32
    src = edge_index[0]
    dst = edge_index[1]
    xp = jnp.concatenate(
        [x, jnp.ones((N, 1), f32), jnp.zeros((N, 124), f32)], axis=1)
    xp = jnp.concatenate([xp, jnp.zeros((NP - N, 128), f32)], axis=0)
    z128 = jnp.zeros((NP, 128), f32)

    pad125 = jnp.zeros((125, 256), f32)
    wl1_t = jnp.concatenate([Wl1.T, pad125], axis=0)
    wr1_t = jnp.concatenate([Wr1.T, pad125], axis=0)

    part = _sc_agg16(xp, src, dst, z128)
    h1 = _tc_layer1(part, xp, wl1_t, bl1.reshape(1, 256), wr1_t)
    agg2 = _sc_agg128(h1, src, dst, z128)
    h2 = _tc_layer(agg2, part, h1, Wl2.T, bl2.reshape(1, 256), Wr2.T)
    agg3 = _sc_agg128(h2, src, dst, z128)
    a_sp, b_sp = _tc_layer(agg3, part, h2, Wl3.T, bl3.reshape(1, 256),
                           Wr3.T, ab=(We1[:, :256].T, We1[:, 256:512].T))
    ga, gb = _sc_edge_gather(a_sp, b_sp, src, dst)
    out = _tc_edge_mlp(
        ga, gb, edge_attr, target_dist.reshape(1, 1),
        We1[:, 512:513].T, We1[:, 513:514].T, be1.reshape(1, 256),
        We2.T, be2.reshape(1, 128), We3, be3.reshape(1, 1))
    return out.reshape(E)
